# baseline (speedup 1.0000x reference)
import jax
import jax.numpy as jnp
from jax.experimental import pallas as pl
from jax.experimental.pallas import tpu as pltpu

_C_IN = 7
_C_OUT = 4
_K = 5
_H = 64
_W = 64
_HP = _H + 2 * (_K - 1)
_WP = _W + 2 * (_K - 1)
_HO = _H + _K - 1
_WO = _W + _K - 1
_CIW = _C_IN * _WP
_KK = _K * _CIW
_NO = _C_OUT * _WO
_MR = 72
_NB = 32


def _body(r_ref, b_ref, x_ref, o_ref, xp_ref, l_ref, p_ref):
    for nb in range(_NB):
        seg = nb * _CIW
        xp_ref[:, pl.ds(seg, _CIW)] = jnp.zeros((_HP, _CIW), jnp.bfloat16)
        for ci in range(_C_IN):
            xp_ref[_K - 1:_K - 1 + _H,
                   pl.ds(seg + ci * _WP + _K - 1, _W)] = (
                x_ref[nb, ci].astype(jnp.bfloat16))

        row = nb * _MR
        for kh in range(_K):
            l_ref[pl.ds(row, _HO), pl.ds(kh * _CIW, _CIW)] = (
                xp_ref[pl.ds(kh, _HO), pl.ds(seg, _CIW)])
        l_ref[pl.ds(row + _HO, _MR - _HO), :] = jnp.zeros(
            (_MR - _HO, _KK), jnp.bfloat16)

    p_ref[...] = jnp.dot(
        l_ref[...], r_ref[...], preferred_element_type=jnp.float32)

    for nb in range(_NB):
        v = p_ref[pl.ds(nb * _MR, _HO), :] + b_ref[0:1, :]
        inner = v * (1.0 + 0.044715 * (v * v)) * 0.7978845608028654
        g = (0.5 * v * (jnp.tanh(inner) + 1.0)).astype(o_ref.dtype)
        for co in range(_C_OUT):
            o_ref[nb, co] = g[:, co * _WO:(co + 1) * _WO]


def _build_banded_weights(weight, bias):
    wf = weight[:, :, ::-1, ::-1]
    wf_t = jnp.transpose(wf, (2, 0, 3, 1))
    band = (jnp.arange(_WP)[:, None] - jnp.arange(_WO)[None, :])
    sel = jnp.clip(band, 0, _K - 1)
    r = wf_t[:, :, sel, :]
    mask = ((band >= 0) & (band < _K))[None, None, :, :, None]
    r = jnp.where(mask, r, 0.0)
    r = jnp.transpose(r, (0, 1, 2, 4, 3))
    r = r.reshape(_KK, _NO).astype(jnp.bfloat16)
    b_rows = jnp.broadcast_to(
        jnp.repeat(bias, _WO)[None, :], (8, _NO)).astype(jnp.float32)
    return r, b_rows


@jax.jit
def _run(x_nchw, weight, bias):
    n = x_nchw.shape[0]
    r_mat, b_rows = _build_banded_weights(weight, bias)

    out = pl.pallas_call(
        _body,
        out_shape=jax.ShapeDtypeStruct((n, _C_OUT, _HO, _WO), jnp.float32),
        grid=(n // _NB,),
        in_specs=[
            pl.BlockSpec((_KK, _NO), lambda i: (0, 0)),
            pl.BlockSpec((8, _NO), lambda i: (0, 0)),
            pl.BlockSpec((_NB, _C_IN, _H, _W), lambda i: (i, 0, 0, 0)),
        ],
        out_specs=pl.BlockSpec(
            (_NB, _C_OUT, _HO, _WO), lambda i: (i, 0, 0, 0)),
        scratch_shapes=[
            pltpu.VMEM((_HP, _NB * _CIW), jnp.bfloat16),
            pltpu.VMEM((_NB * _MR, _KK), jnp.bfloat16),
            pltpu.VMEM((_NB * _MR, _NO), jnp.float32),
        ],
        compiler_params=pltpu.CompilerParams(
            dimension_semantics=("arbitrary",)),
    )(r_mat, b_rows, x_nchw)

    return out


def kernel(x_nchw, weight, bias):
    return _run(x_nchw, weight, bias)


# speedup vs baseline: 1.1384x; 1.1384x over previous
import jax
import jax.numpy as jnp
from jax.experimental import pallas as pl
from jax.experimental.pallas import tpu as pltpu

_C_IN = 7
_C_OUT = 4
_K = 5
_CP = 8
_H = 64
_W = 64
_HP = _H + 2 * (_K - 1)
_WP = _W + 2 * (_K - 1)
_HO = _H + _K - 1
_WO = _W + _K - 1
_L_OUT = _HO * _WP
_SEG = 4992
_L_IN = 5376
_KR = _K * _CP
_NB = 8


def _body(w_ref, b_ref, x_ref, o_ref, xp_ref, xs_ref, p_ref):
    for nb in range(_NB):
        xcol = nb * _L_IN
        xp_ref[:, pl.ds(xcol, _L_IN)] = jnp.zeros((_CP, _L_IN), jnp.bfloat16)
        for h in range(_H):
            dst = xcol + (h + _K - 1) * _WP + (_K - 1)
            xp_ref[0:_C_IN, pl.ds(dst, _W)] = x_ref[
                nb, :, pl.ds(h * _W, _W)].astype(jnp.bfloat16)

        col = nb * _SEG
        for kh in range(_K):
            xs_ref[pl.ds(kh * _CP, _CP), pl.ds(col, _SEG)] = (
                xp_ref[:, pl.ds(xcol + kh * _WP, _SEG)])

        p_ref[:, pl.ds(col, _SEG)] = jnp.dot(
            w_ref[...], xs_ref[:, pl.ds(col, _SEG)],
            preferred_element_type=jnp.float32)

    for nb in range(_NB):
        col = nb * _SEG
        v = p_ref[0:_CP, pl.ds(col, _L_OUT)] + b_ref[:, 0:1]
        for kw in range(1, _K):
            v = v + p_ref[pl.ds(kw * _CP, _CP), pl.ds(col + kw, _L_OUT)]

        inner = v * (0.7978845608028654 + 0.035677408136300125 * (v * v))
        g = 0.5 * v * (jnp.tanh(inner) + 1.0)
        o_ref[nb] = g[:_C_OUT].astype(o_ref.dtype)


def _build_weight_mat(weight, bias):
    w_flip = weight[:, :, ::-1, ::-1]
    w_flip = jnp.pad(
        w_flip, ((0, _CP - _C_IN), (0, _CP - _C_OUT), (0, 0), (0, 0)))
    arr = jnp.transpose(w_flip, (3, 1, 2, 0))
    w_mat = arr.reshape(_KR, _KR)
    b_col = jnp.pad(bias, (0, _CP - _C_OUT))
    b_mat = jnp.broadcast_to(b_col[:, None], (_CP, 128)).astype(jnp.float32)
    return w_mat.astype(jnp.bfloat16), b_mat


@jax.jit
def _run(x_nchw, weight, bias):
    n = x_nchw.shape[0]
    x_flat = x_nchw.reshape(n, _C_IN, _H * _W)
    w_mat, b_mat = _build_weight_mat(weight, bias)

    out = pl.pallas_call(
        _body,
        out_shape=jax.ShapeDtypeStruct((n, _C_OUT, _L_OUT), jnp.float32),
        grid=(n // _NB,),
        in_specs=[
            pl.BlockSpec((_KR, _KR), lambda i: (0, 0)),
            pl.BlockSpec((_CP, 128), lambda i: (0, 0)),
            pl.BlockSpec((_NB, _C_IN, _H * _W), lambda i: (i, 0, 0)),
        ],
        out_specs=pl.BlockSpec((_NB, _C_OUT, _L_OUT), lambda i: (i, 0, 0)),
        scratch_shapes=[
            pltpu.VMEM((_CP, _NB * _L_IN), jnp.bfloat16),
            pltpu.VMEM((_KR, _NB * _SEG), jnp.bfloat16),
            pltpu.VMEM((_KR, _NB * _SEG), jnp.float32),
        ],
        compiler_params=pltpu.CompilerParams(
            dimension_semantics=("parallel",)),
    )(w_mat, b_mat, x_flat)

    y = out.reshape(n, _C_OUT, _HO, _WP)
    return y[:, :, :, :_WO]


def kernel(x_nchw, weight, bias):
    return _run(x_nchw, weight, bias)
